# baseline (speedup 1.0000x reference)
import jax
import jax.numpy as jnp
from jax.experimental import pallas as pl

_DMA_BLOCK = 512
_SUB = 2


def _router_body(x_ref, w_ref, b_ref, o_ref):
    i = pl.program_id(0)
    sub = i % _SUB
    rows = _DMA_BLOCK // _SUB
    o_ref[...] = jax.lax.dot_general(
        x_ref[pl.ds(sub * rows, rows), :], w_ref[...],
        dimension_numbers=(((1,), (1,)), ((), ())),
        preferred_element_type=jnp.float32,
    ) + b_ref[...]


def kernel(x, W, b):
    tokens, d = x.shape
    n_experts = W.shape[0]
    rows = _DMA_BLOCK // _SUB
    return pl.pallas_call(
        _router_body,
        grid=(tokens // rows,),
        in_specs=[
            pl.BlockSpec((_DMA_BLOCK, d), lambda i: (i // _SUB, 0)),
            pl.BlockSpec((n_experts, d), lambda i: (0, 0)),
            pl.BlockSpec((1, n_experts), lambda i: (0, 0)),
        ],
        out_specs=pl.BlockSpec((rows, n_experts), lambda i: (i, 0)),
        out_shape=jax.ShapeDtypeStruct((tokens, n_experts), jnp.float32),
    )(x, W, b.reshape(1, n_experts))


# speedup vs baseline: 1.6113x; 1.6113x over previous
import jax
import jax.numpy as jnp
from jax.experimental import pallas as pl

_TOKEN_BLOCK = 512
_KSPLIT = 4


def _router_body(x_ref, w_ref, b_ref, o_ref):
    d = x_ref.shape[1]
    kc = d // _KSPLIT
    dn = (((1,), (1,)), ((), ()))
    acc = b_ref[...]
    for k in range(_KSPLIT):
        acc = acc + jax.lax.dot_general(
            x_ref[:, pl.ds(k * kc, kc)], w_ref[:, pl.ds(k * kc, kc)],
            dimension_numbers=dn, preferred_element_type=jnp.float32)
    o_ref[...] = acc


def kernel(x, W, b):
    tokens, d = x.shape
    n_experts = W.shape[0]
    blk = _TOKEN_BLOCK
    return pl.pallas_call(
        _router_body,
        grid=(tokens // blk,),
        in_specs=[
            pl.BlockSpec((blk, d), lambda i: (i, 0)),
            pl.BlockSpec((n_experts, d), lambda i: (0, 0)),
            pl.BlockSpec((1, n_experts), lambda i: (0, 0)),
        ],
        out_specs=pl.BlockSpec((blk, n_experts), lambda i: (i, 0)),
        out_shape=jax.ShapeDtypeStruct((tokens, n_experts), jnp.float32),
    )(x, W, b.reshape(1, n_experts))
